# baseline (speedup 1.0000x reference)
import functools
import math

import jax
import jax.numpy as jnp
from jax.experimental import pallas as pl
from jax.experimental.pallas import tpu as pltpu

H = 16
W_BLK = 128
TOK = 256
NSUB = TOK // W_BLK
CVT_ROWS = 256

_TRANS = (((1,), (1,)), ((), ()))


def _fused_attn_kernel(x0_ref, x_ref, wq_hbm, wk_hbm, wv_hbm, wo_hbm, out_ref,
                       wq_s, wk_s, wv_s, wo_s, stg, q_s, k_s, v_s,
                       s_scr, o_scr, sems, *, inv_scale, d, nsteps):
    i = pl.program_id(0)
    nch = d // CVT_ROWS
    srcs = (wq_hbm, wk_hbm, wv_hbm, wo_hbm)
    dsts = (wq_s, wk_s, wv_s, wo_s)
    ntot = 4 * nch

    def project(xv, buf):
        q_s[buf] = jax.lax.dot_general(
            xv, wq_s[...], _TRANS,
            preferred_element_type=jnp.float32).astype(jnp.bfloat16)
        k_s[buf] = jax.lax.dot_general(
            xv, wk_s[...], _TRANS,
            preferred_element_type=jnp.float32).astype(jnp.bfloat16)
        v_s[buf] = jax.lax.dot_general(
            xv, wv_s[...], _TRANS,
            preferred_element_type=jnp.float32).astype(jnp.bfloat16)

    @pl.when(i == 0)
    def _convert_and_prime():
        def dma(t, buf):
            w, c = divmod(t, nch)
            return pltpu.make_async_copy(
                srcs[w].at[pl.ds(c * CVT_ROWS, CVT_ROWS), :],
                stg.at[buf], sems.at[buf])

        dma(0, 0).start()
        for t in range(ntot):
            buf = t % 2
            if t + 1 < ntot:
                dma(t + 1, 1 - buf).start()
            dma(t, buf).wait()
            w, c = divmod(t, nch)
            dsts[w][c * CVT_ROWS:(c + 1) * CVT_ROWS, :] = (
                stg[buf].astype(jnp.bfloat16))
        project(x0_ref[...].astype(jnp.bfloat16), 0)

    @pl.when(i + 1 < nsteps)
    def _project_next():
        project(x_ref[...].astype(jnp.bfloat16), (i + 1) % 2)

    cur = i % 2

    for h in range(H):
        cs = slice(h * W_BLK, (h + 1) * W_BLK)
        for j in range(NSUB):
            rs = slice(j * W_BLK, (j + 1) * W_BLK)
            b = h * NSUB + j
            s_scr[b * W_BLK:(b + 1) * W_BLK, :] = jax.lax.dot_general(
                q_s[cur, rs, cs], k_s[cur, rs, cs], _TRANS,
                preferred_element_type=jnp.float32)

    sv = s_scr[...]
    sv = (sv - jnp.max(sv, axis=-1, keepdims=True)) * inv_scale
    p = jnp.exp(sv)
    p = (p / jnp.sum(p, axis=-1, keepdims=True)).astype(jnp.bfloat16)

    for h in range(H):
        cs = slice(h * W_BLK, (h + 1) * W_BLK)
        for j in range(NSUB):
            rs = slice(j * W_BLK, (j + 1) * W_BLK)
            b = h * NSUB + j
            o_scr[rs, cs] = jnp.dot(
                p[b * W_BLK:(b + 1) * W_BLK, :], v_s[cur, rs, cs],
                preferred_element_type=jnp.float32).astype(jnp.bfloat16)

    out_ref[...] = jax.lax.dot_general(o_scr[...], wo_s[...], _TRANS,
                                       preferred_element_type=jnp.float32)


def kernel(x, Wq, Wk, Wv, Wo):
    B_, T_, D_ = x.shape
    N = B_ * T_
    Dh = D_ // H
    inv_scale = 1.0 / math.sqrt(Dh)
    nsteps = N // TOK

    x2 = x.reshape(N, D_)
    body = functools.partial(_fused_attn_kernel, inv_scale=inv_scale, d=D_,
                             nsteps=nsteps)
    out = pl.pallas_call(
        body,
        grid=(nsteps,),
        in_specs=[
            pl.BlockSpec((TOK, D_), lambda i: (0, 0)),
            pl.BlockSpec((TOK, D_),
                         lambda i: (jnp.minimum(i + 1, nsteps - 1), 0)),
            pl.BlockSpec(memory_space=pl.ANY),
            pl.BlockSpec(memory_space=pl.ANY),
            pl.BlockSpec(memory_space=pl.ANY),
            pl.BlockSpec(memory_space=pl.ANY),
        ],
        out_specs=pl.BlockSpec((TOK, D_), lambda i: (i, 0)),
        out_shape=jax.ShapeDtypeStruct((N, D_), jnp.float32),
        scratch_shapes=[
            pltpu.VMEM((D_, D_), jnp.bfloat16),
            pltpu.VMEM((D_, D_), jnp.bfloat16),
            pltpu.VMEM((D_, D_), jnp.bfloat16),
            pltpu.VMEM((D_, D_), jnp.bfloat16),
            pltpu.VMEM((2, CVT_ROWS, D_), jnp.float32),
            pltpu.VMEM((2, TOK, D_), jnp.bfloat16),
            pltpu.VMEM((2, TOK, D_), jnp.bfloat16),
            pltpu.VMEM((2, TOK, D_), jnp.bfloat16),
            pltpu.VMEM((H * NSUB * W_BLK, W_BLK), jnp.float32),
            pltpu.VMEM((TOK, D_), jnp.bfloat16),
            pltpu.SemaphoreType.DMA((2,)),
        ],
        compiler_params=pltpu.CompilerParams(
            dimension_semantics=("arbitrary",),
        ),
    )(x2, x2, Wq, Wk, Wv, Wo)
    return out.reshape(B_, T_, D_)
